# Initial kernel scaffold; baseline (speedup 1.0000x reference)
import jax
import jax.numpy as jnp
from jax.experimental import pallas as pl


def kernel(output, target):
    raise NotImplementedError("write your pallas kernel here")


# speedup vs baseline: 2.3228x; 2.3228x over previous
import math

import jax
import jax.numpy as jnp
from jax.experimental import pallas as pl

_LS = 0.1
_CONF = 1.0 - _LS
_EPS = 1e-12
_V = 100000
_B = 1024
_G = _V - 100
_S = _LS / (_V - 1)
_LOG_S = math.log(_S)
_LOG_C = math.log(_CONF)
_R = 32
_NB = _B // _R


def _main_kernel(x_ref, t_ref, out_ref):
    i = pl.program_id(0)
    x = x_ref[...]
    t = t_ref[...]
    l = jnp.log(x + _EPS)
    s_row = jnp.sum(l, axis=1, keepdims=True)
    cols = jax.lax.broadcasted_iota(jnp.int32, (_R, _V), 1)
    lt = jnp.sum(jnp.where(cols == t, l, 0.0), axis=1, keepdims=True)
    lg = l[:, _G:_G + 1]
    delta = (t != _G).astype(jnp.float32)
    valid = (t != -100).astype(jnp.float32)
    const = _S * _LOG_S * (_V - 1 - delta) + _CONF * _LOG_C
    row = -_S * s_row + (_S - _CONF) * lt + _S * delta * lg + const
    partial = jnp.sum(valid * row)

    @pl.when(i == 0)
    def _init():
        out_ref[...] = jnp.zeros((1, 1), jnp.float32)

    out_ref[...] += jnp.reshape(partial, (1, 1))


def kernel(output, target):
    out = pl.pallas_call(
        _main_kernel,
        grid=(_NB,),
        in_specs=[
            pl.BlockSpec((_R, _V), lambda i: (i, 0)),
            pl.BlockSpec((_R, 1), lambda i: (i, 0)),
        ],
        out_specs=pl.BlockSpec((1, 1), lambda i: (0, 0)),
        out_shape=jax.ShapeDtypeStruct((1, 1), jnp.float32),
    )(output, target.reshape(_B, 1))
    return out[0, 0]


# speedup vs baseline: 2.3770x; 1.0233x over previous
import math

import jax
import jax.numpy as jnp
from jax.experimental import pallas as pl

_LS = 0.1
_CONF = 1.0 - _LS
_EPS = 1e-12
_V = 100000
_B = 1024
_G = _V - 100
_S = _LS / (_V - 1)
_LOG_S = math.log(_S)
_LOG_C = math.log(_CONF)
_R = 64
_NB = _B // _R


def _main_kernel(x_ref, t_ref, out_ref):
    i = pl.program_id(0)
    x = x_ref[...]
    t = t_ref[...]
    l = jnp.log(x + _EPS)
    cols = jax.lax.broadcasted_iota(jnp.int32, (_R, _V), 1)
    valid = (t != -100).astype(jnp.float32)
    c = jnp.where(cols == t, -_CONF, -_S) * valid
    delta = (t != _G).astype(jnp.float32)
    lg = jnp.log(x[:, _G:_G + 1] + _EPS)
    const = _S * _LOG_S * (_V - 1 - delta) + _CONF * _LOG_C
    partial = jnp.sum(c * l) + jnp.sum(valid * (_S * delta * lg + const))

    @pl.when(i == 0)
    def _init():
        out_ref[...] = jnp.zeros((1, 1), jnp.float32)

    out_ref[...] += jnp.reshape(partial, (1, 1))


def kernel(output, target):
    out = pl.pallas_call(
        _main_kernel,
        grid=(_NB,),
        in_specs=[
            pl.BlockSpec((_R, _V), lambda i: (i, 0)),
            pl.BlockSpec((_R, 1), lambda i: (i, 0)),
        ],
        out_specs=pl.BlockSpec((1, 1), lambda i: (0, 0)),
        out_shape=jax.ShapeDtypeStruct((1, 1), jnp.float32),
    )(output, target.reshape(_B, 1))
    return out[0, 0]
